# baseline (speedup 1.0000x reference)
import functools

import jax
import jax.numpy as jnp
from jax import lax
from jax.experimental import pallas as pl
from jax.experimental.pallas import tpu as pltpu
from jax.experimental.pallas import tpu_sc as plsc

N = 10000
E = 160000
D = 256
H = 256
OUT = 128

NC = 2
NS = 16
NP = 10240
SLAB = NP // NS
EPT = E // NS
B = 125
NB = EPT // B
KB = 16
NCH = NB // KB
EPT_DEG = E // (NC * NS)
BD = 125
NBD = EPT_DEG // BD


def _vector_mesh():
    return plsc.VectorSubcoreMesh(core_axis_name="c", subcore_axis_name="s")


def _deg_call(dstd, ones_rows, zslab16):
    @functools.partial(
        pl.kernel,
        out_type=[jax.ShapeDtypeStruct((NP, 128), jnp.float32)] * 2,
        mesh=_vector_mesh(),
        scratch_types=[
            pltpu.VMEM((NBD, BD), jnp.int32),
            pltpu.VMEM((BD, 128), jnp.float32),
            pltpu.VMEM_SHARED((NP, 128), jnp.float32),
            pltpu.SemaphoreType.DMA,
        ],
    )
    def k(dstd_hbm, ones_hbm, z_hbm, outA_hbm, outB_hbm, idx_d, ones_v, acc_sh,
          sem):
        c = lax.axis_index("c")
        s = lax.axis_index("s")

        def run(chunk, out_hbm):
            pltpu.sync_copy(z_hbm, acc_sh.at[pl.ds(s * SLAB, SLAB)])
            pltpu.sync_copy(dstd_hbm.at[chunk], idx_d)
            pltpu.sync_copy(ones_hbm, ones_v)
            plsc.subcore_barrier()

            @pl.loop(0, NBD)
            def _(j):
                pltpu.sync_copy(ones_v, acc_sh.at[idx_d.at[j]], add=True)

            plsc.subcore_barrier()
            pltpu.sync_copy(acc_sh.at[pl.ds(s * SLAB, SLAB)],
                            out_hbm.at[pl.ds(s * SLAB, SLAB)])

        @pl.when(c == 0)
        def _():
            run(s, outA_hbm)

        @pl.when(c == 1)
        def _():
            run(s + NS, outB_hbm)

    return k(dstd, ones_rows, zslab16)


def _msgpass_call(ztA, ztB, srcr, dstr, zslab):
    @functools.partial(
        pl.kernel,
        out_type=[jax.ShapeDtypeStruct((NP, 128), jnp.float32)] * 2,
        mesh=_vector_mesh(),
        scratch_types=[
            pltpu.VMEM((KB, B), jnp.int32),
            pltpu.VMEM((KB, B), jnp.int32),
            pltpu.VMEM((B, 128), jnp.float32),
            pltpu.VMEM((B, 128), jnp.float32),
            pltpu.VMEM_SHARED((NP, 128), jnp.float32),
            pltpu.SemaphoreType.DMA,
            pltpu.SemaphoreType.DMA,
            pltpu.SemaphoreType.DMA,
            pltpu.SemaphoreType.DMA,
        ],
    )
    def k(ztA_hbm, ztB_hbm, src_hbm, dst_hbm, z_hbm, outA_hbm, outB_hbm,
          idx_s, idx_d, rows0, rows1, acc_sh, sem0, sem1, sem2, sem3):
        c = lax.axis_index("c")
        s = lax.axis_index("s")

        def run(zt_hbm, out_hbm):
            pltpu.sync_copy(z_hbm, acc_sh.at[pl.ds(s * SLAB, SLAB)])
            plsc.subcore_barrier()

            def start_g(j, buf, sem):
                pltpu.make_async_copy(zt_hbm.at[idx_s.at[j]], buf, sem).start()

            def wait_g(buf, sem):
                pltpu.make_async_copy(zt_hbm.at[idx_s.at[0]], buf, sem).wait()

            def start_sc(j, buf, sem):
                pltpu.async_copy(buf, acc_sh.at[idx_d.at[j]], sem, add=True)

            def wait_sc(buf, sem):
                pltpu.make_async_copy(buf, acc_sh.at[idx_d.at[0]], sem).wait()

            @pl.loop(0, NCH)
            def _(ch):
                pltpu.sync_copy(src_hbm.at[s].at[pl.ds(ch * KB, KB)], idx_s)
                pltpu.sync_copy(dst_hbm.at[s].at[pl.ds(ch * KB, KB)], idx_d)
                start_g(0, rows0, sem0)
                start_g(1, rows1, sem1)

                @pl.loop(0, KB // 2)
                def _(j2):
                    j = 2 * j2
                    wait_g(rows0, sem0)
                    start_sc(j, rows0, sem2)
                    wait_g(rows1, sem1)
                    start_sc(j + 1, rows1, sem3)
                    wait_sc(rows0, sem2)

                    @pl.when(j + 2 < KB)
                    def _():
                        start_g(j + 2, rows0, sem0)

                    wait_sc(rows1, sem3)

                    @pl.when(j + 3 < KB)
                    def _():
                        start_g(j + 3, rows1, sem1)

            plsc.subcore_barrier()
            pltpu.sync_copy(acc_sh.at[pl.ds(s * SLAB, SLAB)],
                            out_hbm.at[pl.ds(s * SLAB, SLAB)])

        @pl.when(c == 0)
        def _():
            run(ztA_hbm, outA_hbm)

        @pl.when(c == 1)
        def _():
            run(ztB_hbm, outB_hbm)

    return k(ztA, ztB, srcr, dstr, zslab)


_BLK = 1000


def _mm0_call(x, W0):
    def body(h_ref, w_ref, z_ref):
        z_ref[...] = jnp.dot(h_ref[...].astype(jnp.bfloat16), w_ref[...],
                             preferred_element_type=jnp.float32)

    return pl.pallas_call(
        body,
        grid=(N // _BLK,),
        in_specs=[pl.BlockSpec((_BLK, D), lambda i: (i, 0)),
                  pl.BlockSpec((D, H), lambda i: (0, 0))],
        out_specs=pl.BlockSpec((_BLK, H), lambda i: (i, 0)),
        out_shape=jax.ShapeDtypeStruct((N, H), jnp.float32),
    )(x, W0)


def _scale0_call(degA, degB, z0):
    def body(da_ref, db_ref, z_ref, dv_ref, za_ref, zb_ref):
        d = da_ref[:, :1] + db_ref[:, :1] + 1.0
        dv = jnp.broadcast_to(lax.rsqrt(d), (_BLK, 128))
        dv_ref[...] = dv
        z = z_ref[...]
        za_ref[...] = z[:, :128] * dv
        zb_ref[...] = z[:, 128:] * dv

    return pl.pallas_call(
        body,
        grid=(N // _BLK,),
        in_specs=[pl.BlockSpec((_BLK, 128), lambda i: (i, 0)),
                  pl.BlockSpec((_BLK, 128), lambda i: (i, 0)),
                  pl.BlockSpec((_BLK, H), lambda i: (i, 0))],
        out_specs=[pl.BlockSpec((_BLK, 128), lambda i: (i, 0))] * 3,
        out_shape=[jax.ShapeDtypeStruct((N, 128), jnp.float32)] * 3,
    )(degA, degB, z0)


def _fused_call(accA, accB, ztA, ztB, dinv, b2d, g2d, beta2d, Wn):
    def body(aa, ab, za, zb, dv_ref, b_ref, g_ref, be_ref, w_ref,
             oa_ref, ob_ref):
        dv = dv_ref[...]
        sc = g_ref[...] * (1.0 / jnp.sqrt(1.0 + 1e-5))
        oA = jnp.maximum(dv * (aa[...] + za[...]) + b_ref[:, :128], 0.0)
        oB = jnp.maximum(dv * (ab[...] + zb[...]) + b_ref[:, 128:], 0.0)
        h = jnp.concatenate([oA, oB], axis=1) * sc + be_ref[...]
        z = jnp.dot(h.astype(jnp.bfloat16), w_ref[...],
                    preferred_element_type=jnp.float32)
        oa_ref[...] = z[:, :128] * dv
        ob_ref[...] = z[:, 128:] * dv

    return pl.pallas_call(
        body,
        grid=(N // _BLK,),
        in_specs=[pl.BlockSpec((_BLK, 128), lambda i: (i, 0))] * 5
        + [pl.BlockSpec((1, H), lambda i: (0, 0))] * 3
        + [pl.BlockSpec((H, H), lambda i: (0, 0))],
        out_specs=[pl.BlockSpec((_BLK, 128), lambda i: (i, 0))] * 2,
        out_shape=[jax.ShapeDtypeStruct((N, 128), jnp.float32)] * 2,
    )(accA, accB, ztA, ztB, dinv, b2d, g2d, beta2d, Wn)


def _tail_call(accA, accB, ztA, ztB, dinv, b2d, Wp1, bp1_2d, Wp2, bp2_2d):
    grid_n = N // _BLK

    def body(aa, ab, za, zb, dv_ref, b_ref, w1_ref, b1_ref, w2_ref, b2_ref,
             o_ref, ssum, smax):
        i = pl.program_id(0)
        dv = dv_ref[...]
        oA = jnp.maximum(dv * (aa[...] + za[...]) + b_ref[:, :128], 0.0)
        oB = jnp.maximum(dv * (ab[...] + zb[...]) + b_ref[:, 128:], 0.0)
        x = jnp.concatenate([oA, oB], axis=1)
        xr = x.reshape(_BLK // 8, 8, H)
        ps = jnp.sum(xr, axis=0)
        pm = jnp.max(xr, axis=0)

        @pl.when(i == 0)
        def _():
            ssum[...] = ps
            smax[...] = pm

        @pl.when(i > 0)
        def _():
            ssum[...] = ssum[...] + ps
            smax[...] = jnp.maximum(smax[...], pm)

        @pl.when(i == grid_n - 1)
        def _():
            tot = jnp.sum(ssum[...], axis=0, keepdims=True) * (1.0 / N)
            mx = jnp.max(smax[...], axis=0, keepdims=True)
            graph = jnp.concatenate([tot, mx], axis=1)
            z = jnp.maximum(
                jnp.dot(graph, w1_ref[...], preferred_element_type=jnp.float32)
                + b1_ref[...], 0.0)
            o_ref[...] = (
                jnp.dot(z, w2_ref[...], preferred_element_type=jnp.float32)
                + b2_ref[...])

    return pl.pallas_call(
        body,
        grid=(grid_n,),
        in_specs=[pl.BlockSpec((_BLK, 128), lambda i: (i, 0))] * 5
        + [pl.BlockSpec((1, H), lambda i: (0, 0)),
           pl.BlockSpec((2 * H, H), lambda i: (0, 0)),
           pl.BlockSpec((1, H), lambda i: (0, 0)),
           pl.BlockSpec((H, OUT), lambda i: (0, 0)),
           pl.BlockSpec((1, OUT), lambda i: (0, 0))],
        out_specs=pl.BlockSpec((1, OUT), lambda i: (0, 0)),
        out_shape=jax.ShapeDtypeStruct((1, OUT), jnp.float32),
        scratch_shapes=[pltpu.VMEM((8, H), jnp.float32),
                        pltpu.VMEM((8, H), jnp.float32)],
    )(accA, accB, ztA, ztB, dinv, b2d, Wp1, bp1_2d, Wp2, bp2_2d)


def kernel(x, edge_index, W0, b0, W1, b1, W2, b2, g0, beta0, g1, beta1,
           Wp1, bp1, Wp2, bp2):
    src = edge_index[0]
    dst = edge_index[1]
    srcr = src.reshape(NS, NB, B)
    dstr = dst.reshape(NS, NB, B)
    dstd = dst.reshape(NC * NS, NBD, BD)
    zslab = jnp.zeros((SLAB, 128), jnp.float32)
    ones_rows = jnp.ones((BD, 128), jnp.float32)

    degA, degB = _deg_call(dstd, ones_rows, zslab)
    z0 = _mm0_call(x, W0.astype(jnp.bfloat16))
    dinv, ztA, ztB = _scale0_call(degA, degB, z0)

    accA, accB = _msgpass_call(ztA, ztB, srcr, dstr, zslab)
    ztA, ztB = _fused_call(accA, accB, ztA, ztB, dinv,
                           b0.reshape(1, H), g0.reshape(1, H),
                           beta0.reshape(1, H), W1.astype(jnp.bfloat16))
    accA, accB = _msgpass_call(ztA, ztB, srcr, dstr, zslab)
    ztA, ztB = _fused_call(accA, accB, ztA, ztB, dinv,
                           b1.reshape(1, H), g1.reshape(1, H),
                           beta1.reshape(1, H), W2.astype(jnp.bfloat16))
    accA, accB = _msgpass_call(ztA, ztB, srcr, dstr, zslab)

    return _tail_call(accA, accB, ztA, ztB, dinv, b2.reshape(1, H),
                      Wp1, bp1.reshape(1, H), Wp2, bp2.reshape(1, OUT))


# speedup vs baseline: 1.0678x; 1.0678x over previous
import functools

import jax
import jax.numpy as jnp
from jax import lax
from jax.experimental import pallas as pl
from jax.experimental.pallas import tpu as pltpu
from jax.experimental.pallas import tpu_sc as plsc

N = 10000
E = 160000
D = 256
H = 256
OUT = 128

NC = 2
NS = 16
NP = 10240
SLAB = NP // NS
EPT = E // NS
B = 125
NB = EPT // B
KB = 16
NCH = NB // KB
EPT_DEG = E // (NC * NS)
BD = 125
NBD = EPT_DEG // BD


def _vector_mesh():
    return plsc.VectorSubcoreMesh(core_axis_name="c", subcore_axis_name="s")


def _deg_call(dstd, ones_rows, zslab16):
    @functools.partial(
        pl.kernel,
        out_type=[jax.ShapeDtypeStruct((NP, 128), jnp.float32)] * 2,
        mesh=_vector_mesh(),
        scratch_types=[
            pltpu.VMEM((NBD, BD), jnp.int32),
            pltpu.VMEM((BD, 128), jnp.float32),
            pltpu.VMEM_SHARED((NP, 128), jnp.float32),
            pltpu.SemaphoreType.DMA,
        ],
    )
    def k(dstd_hbm, ones_hbm, z_hbm, outA_hbm, outB_hbm, idx_d, ones_v, acc_sh,
          sem):
        c = lax.axis_index("c")
        s = lax.axis_index("s")

        def run(chunk, out_hbm):
            pltpu.sync_copy(z_hbm, acc_sh.at[pl.ds(s * SLAB, SLAB)])
            pltpu.sync_copy(dstd_hbm.at[chunk], idx_d)
            pltpu.sync_copy(ones_hbm, ones_v)
            plsc.subcore_barrier()

            @pl.loop(0, NBD)
            def _(j):
                pltpu.sync_copy(ones_v, acc_sh.at[idx_d.at[j]], add=True)

            plsc.subcore_barrier()
            pltpu.sync_copy(acc_sh.at[pl.ds(s * SLAB, SLAB)],
                            out_hbm.at[pl.ds(s * SLAB, SLAB)])

        @pl.when(c == 0)
        def _():
            run(s, outA_hbm)

        @pl.when(c == 1)
        def _():
            run(s + NS, outB_hbm)

    return k(dstd, ones_rows, zslab16)


def _msgpass_call(ztA, ztB, srcr, dstr, zslab):
    @functools.partial(
        pl.kernel,
        out_type=[jax.ShapeDtypeStruct((NP, 128), jnp.float32)] * 2,
        mesh=_vector_mesh(),
        scratch_types=[
            pltpu.VMEM((KB, B), jnp.int32),
            pltpu.VMEM((KB, B), jnp.int32),
            pltpu.VMEM((B, 128), jnp.float32),
            pltpu.VMEM((B, 128), jnp.float32),
            pltpu.VMEM_SHARED((NP, 128), jnp.float32),
            pltpu.SemaphoreType.DMA,
            pltpu.SemaphoreType.DMA,
        ],
    )
    def k(ztA_hbm, ztB_hbm, src_hbm, dst_hbm, z_hbm, outA_hbm, outB_hbm,
          idx_s, idx_d, rows0, rows1, acc_sh, sem0, sem1):
        c = lax.axis_index("c")
        s = lax.axis_index("s")

        def run(zt_hbm, out_hbm):
            pltpu.sync_copy(z_hbm, acc_sh.at[pl.ds(s * SLAB, SLAB)])
            plsc.subcore_barrier()

            def start_g(j, buf, sem):
                pltpu.make_async_copy(zt_hbm.at[idx_s.at[j]], buf, sem).start()

            def wait_g(buf, sem):
                pltpu.make_async_copy(zt_hbm.at[idx_s.at[0]], buf, sem).wait()

            @pl.loop(0, NCH)
            def _(ch):
                pltpu.sync_copy(src_hbm.at[s].at[pl.ds(ch * KB, KB)], idx_s)
                pltpu.sync_copy(dst_hbm.at[s].at[pl.ds(ch * KB, KB)], idx_d)
                start_g(0, rows0, sem0)

                @pl.loop(0, KB // 2)
                def _(j2):
                    j = 2 * j2
                    wait_g(rows0, sem0)
                    start_g(j + 1, rows1, sem1)
                    pltpu.sync_copy(rows0, acc_sh.at[idx_d.at[j]], add=True)
                    wait_g(rows1, sem1)

                    @pl.when(j + 2 < KB)
                    def _():
                        start_g(j + 2, rows0, sem0)

                    pltpu.sync_copy(rows1, acc_sh.at[idx_d.at[j + 1]], add=True)

            plsc.subcore_barrier()
            pltpu.sync_copy(acc_sh.at[pl.ds(s * SLAB, SLAB)],
                            out_hbm.at[pl.ds(s * SLAB, SLAB)])

        @pl.when(c == 0)
        def _():
            run(ztA_hbm, outA_hbm)

        @pl.when(c == 1)
        def _():
            run(ztB_hbm, outB_hbm)

    return k(ztA, ztB, srcr, dstr, zslab)


_BLK = 1000


def _mm0_call(x, W0):
    def body(h_ref, w_ref, z_ref):
        z_ref[...] = jnp.dot(h_ref[...].astype(jnp.bfloat16), w_ref[...],
                             preferred_element_type=jnp.float32)

    return pl.pallas_call(
        body,
        grid=(N // _BLK,),
        in_specs=[pl.BlockSpec((_BLK, D), lambda i: (i, 0)),
                  pl.BlockSpec((D, H), lambda i: (0, 0))],
        out_specs=pl.BlockSpec((_BLK, H), lambda i: (i, 0)),
        out_shape=jax.ShapeDtypeStruct((N, H), jnp.float32),
    )(x, W0)


def _scale0_call(degA, degB, z0):
    def body(da_ref, db_ref, z_ref, dv_ref, za_ref, zb_ref):
        d = da_ref[:, :1] + db_ref[:, :1] + 1.0
        dv = jnp.broadcast_to(lax.rsqrt(d), (_BLK, 128))
        dv_ref[...] = dv
        z = z_ref[...]
        za_ref[...] = z[:, :128] * dv
        zb_ref[...] = z[:, 128:] * dv

    return pl.pallas_call(
        body,
        grid=(N // _BLK,),
        in_specs=[pl.BlockSpec((_BLK, 128), lambda i: (i, 0)),
                  pl.BlockSpec((_BLK, 128), lambda i: (i, 0)),
                  pl.BlockSpec((_BLK, H), lambda i: (i, 0))],
        out_specs=[pl.BlockSpec((_BLK, 128), lambda i: (i, 0))] * 3,
        out_shape=[jax.ShapeDtypeStruct((N, 128), jnp.float32)] * 3,
    )(degA, degB, z0)


def _fused_call(accA, accB, ztA, ztB, dinv, b2d, g2d, beta2d, Wn):
    def body(aa, ab, za, zb, dv_ref, b_ref, g_ref, be_ref, w_ref,
             oa_ref, ob_ref):
        dv = dv_ref[...]
        sc = g_ref[...] * (1.0 / jnp.sqrt(1.0 + 1e-5))
        oA = jnp.maximum(dv * (aa[...] + za[...]) + b_ref[:, :128], 0.0)
        oB = jnp.maximum(dv * (ab[...] + zb[...]) + b_ref[:, 128:], 0.0)
        h = jnp.concatenate([oA, oB], axis=1) * sc + be_ref[...]
        z = jnp.dot(h.astype(jnp.bfloat16), w_ref[...],
                    preferred_element_type=jnp.float32)
        oa_ref[...] = z[:, :128] * dv
        ob_ref[...] = z[:, 128:] * dv

    return pl.pallas_call(
        body,
        grid=(N // _BLK,),
        in_specs=[pl.BlockSpec((_BLK, 128), lambda i: (i, 0))] * 5
        + [pl.BlockSpec((1, H), lambda i: (0, 0))] * 3
        + [pl.BlockSpec((H, H), lambda i: (0, 0))],
        out_specs=[pl.BlockSpec((_BLK, 128), lambda i: (i, 0))] * 2,
        out_shape=[jax.ShapeDtypeStruct((N, 128), jnp.float32)] * 2,
    )(accA, accB, ztA, ztB, dinv, b2d, g2d, beta2d, Wn)


def _tail_call(accA, accB, ztA, ztB, dinv, b2d, Wp1, bp1_2d, Wp2, bp2_2d):
    grid_n = N // _BLK

    def body(aa, ab, za, zb, dv_ref, b_ref, w1_ref, b1_ref, w2_ref, b2_ref,
             o_ref, ssum, smax):
        i = pl.program_id(0)
        dv = dv_ref[...]
        oA = jnp.maximum(dv * (aa[...] + za[...]) + b_ref[:, :128], 0.0)
        oB = jnp.maximum(dv * (ab[...] + zb[...]) + b_ref[:, 128:], 0.0)
        x = jnp.concatenate([oA, oB], axis=1)
        xr = x.reshape(_BLK // 8, 8, H)
        ps = jnp.sum(xr, axis=0)
        pm = jnp.max(xr, axis=0)

        @pl.when(i == 0)
        def _():
            ssum[...] = ps
            smax[...] = pm

        @pl.when(i > 0)
        def _():
            ssum[...] = ssum[...] + ps
            smax[...] = jnp.maximum(smax[...], pm)

        @pl.when(i == grid_n - 1)
        def _():
            tot = jnp.sum(ssum[...], axis=0, keepdims=True) * (1.0 / N)
            mx = jnp.max(smax[...], axis=0, keepdims=True)
            graph = jnp.concatenate([tot, mx], axis=1)
            z = jnp.maximum(
                jnp.dot(graph, w1_ref[...], preferred_element_type=jnp.float32)
                + b1_ref[...], 0.0)
            o_ref[...] = (
                jnp.dot(z, w2_ref[...], preferred_element_type=jnp.float32)
                + b2_ref[...])

    return pl.pallas_call(
        body,
        grid=(grid_n,),
        in_specs=[pl.BlockSpec((_BLK, 128), lambda i: (i, 0))] * 5
        + [pl.BlockSpec((1, H), lambda i: (0, 0)),
           pl.BlockSpec((2 * H, H), lambda i: (0, 0)),
           pl.BlockSpec((1, H), lambda i: (0, 0)),
           pl.BlockSpec((H, OUT), lambda i: (0, 0)),
           pl.BlockSpec((1, OUT), lambda i: (0, 0))],
        out_specs=pl.BlockSpec((1, OUT), lambda i: (0, 0)),
        out_shape=jax.ShapeDtypeStruct((1, OUT), jnp.float32),
        scratch_shapes=[pltpu.VMEM((8, H), jnp.float32),
                        pltpu.VMEM((8, H), jnp.float32)],
    )(accA, accB, ztA, ztB, dinv, b2d, Wp1, bp1_2d, Wp2, bp2_2d)


def kernel(x, edge_index, W0, b0, W1, b1, W2, b2, g0, beta0, g1, beta1,
           Wp1, bp1, Wp2, bp2):
    src = edge_index[0]
    dst = edge_index[1]
    srcr = src.reshape(NS, NB, B)
    dstr = dst.reshape(NS, NB, B)
    dstd = dst.reshape(NC * NS, NBD, BD)
    zslab = jnp.zeros((SLAB, 128), jnp.float32)
    ones_rows = jnp.ones((BD, 128), jnp.float32)

    degA, degB = _deg_call(dstd, ones_rows, zslab)
    z0 = _mm0_call(x, W0.astype(jnp.bfloat16))
    dinv, ztA, ztB = _scale0_call(degA, degB, z0)

    accA, accB = _msgpass_call(ztA, ztB, srcr, dstr, zslab)
    ztA, ztB = _fused_call(accA, accB, ztA, ztB, dinv,
                           b0.reshape(1, H), g0.reshape(1, H),
                           beta0.reshape(1, H), W1.astype(jnp.bfloat16))
    accA, accB = _msgpass_call(ztA, ztB, srcr, dstr, zslab)
    ztA, ztB = _fused_call(accA, accB, ztA, ztB, dinv,
                           b1.reshape(1, H), g1.reshape(1, H),
                           beta1.reshape(1, H), W2.astype(jnp.bfloat16))
    accA, accB = _msgpass_call(ztA, ztB, srcr, dstr, zslab)

    return _tail_call(accA, accB, ztA, ztB, dinv, b2.reshape(1, H),
                      Wp1, bp1.reshape(1, H), Wp2, bp2.reshape(1, OUT))


# speedup vs baseline: 1.0815x; 1.0128x over previous
import functools

import jax
import jax.numpy as jnp
from jax import lax
from jax.experimental import pallas as pl
from jax.experimental.pallas import tpu as pltpu
from jax.experimental.pallas import tpu_sc as plsc

N = 10000
E = 160000
D = 256
H = 256
OUT = 128

NC = 2
NS = 16
NP = 10240
SLAB = NP // NS
EPT = E // NS
B = 125
NB = EPT // B
KB = 16
NCH = NB // KB
EPT_DEG = E // (NC * NS)
BD = 125
NBD = EPT_DEG // BD


def _vector_mesh():
    return plsc.VectorSubcoreMesh(core_axis_name="c", subcore_axis_name="s")


def _deg_call(dstd, ones_rows, zslab16):
    @functools.partial(
        pl.kernel,
        out_type=[jax.ShapeDtypeStruct((NP, 128), jnp.float32)] * 2,
        mesh=_vector_mesh(),
        scratch_types=[
            pltpu.VMEM((NBD, BD), jnp.int32),
            pltpu.VMEM((BD, 128), jnp.float32),
            pltpu.VMEM_SHARED((NP, 128), jnp.float32),
            pltpu.SemaphoreType.DMA,
        ],
    )
    def k(dstd_hbm, ones_hbm, z_hbm, outA_hbm, outB_hbm, idx_d, ones_v, acc_sh,
          sem):
        c = lax.axis_index("c")
        s = lax.axis_index("s")

        def run(chunk, out_hbm):
            pltpu.sync_copy(z_hbm, acc_sh.at[pl.ds(s * SLAB, SLAB)])
            pltpu.sync_copy(dstd_hbm.at[chunk], idx_d)
            pltpu.sync_copy(ones_hbm, ones_v)
            plsc.subcore_barrier()

            @pl.loop(0, NBD)
            def _(j):
                pltpu.sync_copy(ones_v, acc_sh.at[idx_d.at[j]], add=True)

            plsc.subcore_barrier()
            pltpu.sync_copy(acc_sh.at[pl.ds(s * SLAB, SLAB)],
                            out_hbm.at[pl.ds(s * SLAB, SLAB)])

        @pl.when(c == 0)
        def _():
            run(s, outA_hbm)

        @pl.when(c == 1)
        def _():
            run(s + NS, outB_hbm)

    return k(dstd, ones_rows, zslab16)


def _msgpass_call(ztA, ztB, srcr, dstr, zslab):
    @functools.partial(
        pl.kernel,
        out_type=[jax.ShapeDtypeStruct((NP, 128), jnp.float32)] * 2,
        mesh=_vector_mesh(),
        scratch_types=[
            pltpu.VMEM((KB, B), jnp.int32),
            pltpu.VMEM((KB, B), jnp.int32),
            pltpu.VMEM((B, 128), jnp.float32),
            pltpu.VMEM((B, 128), jnp.float32),
            pltpu.VMEM_SHARED((NP, 128), jnp.float32),
            pltpu.SemaphoreType.DMA,
            pltpu.SemaphoreType.DMA,
            pltpu.SemaphoreType.DMA,
        ],
    )
    def k(ztA_hbm, ztB_hbm, src_hbm, dst_hbm, z_hbm, outA_hbm, outB_hbm,
          idx_s, idx_d, rows0, rows1, acc_sh, sem0, sem1, semz):
        c = lax.axis_index("c")
        s = lax.axis_index("s")

        def run(zt_hbm, out_hbm):
            zdesc = pltpu.async_copy(z_hbm, acc_sh.at[pl.ds(s * SLAB, SLAB)],
                                     semz)

            def start_g(j, buf, sem):
                pltpu.make_async_copy(zt_hbm.at[idx_s.at[j]], buf, sem).start()

            def wait_g(buf, sem):
                pltpu.make_async_copy(zt_hbm.at[idx_s.at[0]], buf, sem).wait()

            @pl.loop(0, NCH)
            def _(ch):
                pltpu.sync_copy(src_hbm.at[s].at[pl.ds(ch * KB, KB)], idx_s)
                pltpu.sync_copy(dst_hbm.at[s].at[pl.ds(ch * KB, KB)], idx_d)
                start_g(0, rows0, sem0)

                @pl.when(ch == 0)
                def _():
                    zdesc.wait()
                    plsc.subcore_barrier()

                @pl.loop(0, KB // 2)
                def _(j2):
                    j = 2 * j2
                    wait_g(rows0, sem0)
                    start_g(j + 1, rows1, sem1)
                    pltpu.sync_copy(rows0, acc_sh.at[idx_d.at[j]], add=True)
                    wait_g(rows1, sem1)

                    @pl.when(j + 2 < KB)
                    def _():
                        start_g(j + 2, rows0, sem0)

                    pltpu.sync_copy(rows1, acc_sh.at[idx_d.at[j + 1]], add=True)

            plsc.subcore_barrier()
            pltpu.sync_copy(acc_sh.at[pl.ds(s * SLAB, SLAB)],
                            out_hbm.at[pl.ds(s * SLAB, SLAB)])

        @pl.when(c == 0)
        def _():
            run(ztA_hbm, outA_hbm)

        @pl.when(c == 1)
        def _():
            run(ztB_hbm, outB_hbm)

    return k(ztA, ztB, srcr, dstr, zslab)


_BLK = 1000


def _mm0_call(x, W0):
    def body(h_ref, w_ref, z_ref):
        z_ref[...] = jnp.dot(h_ref[...].astype(jnp.bfloat16), w_ref[...],
                             preferred_element_type=jnp.float32)

    return pl.pallas_call(
        body,
        grid=(N // _BLK,),
        in_specs=[pl.BlockSpec((_BLK, D), lambda i: (i, 0)),
                  pl.BlockSpec((D, H), lambda i: (0, 0))],
        out_specs=pl.BlockSpec((_BLK, H), lambda i: (i, 0)),
        out_shape=jax.ShapeDtypeStruct((N, H), jnp.float32),
    )(x, W0)


def _scale0_call(degA, degB, z0):
    def body(da_ref, db_ref, z_ref, dv_ref, za_ref, zb_ref):
        d = da_ref[:, :1] + db_ref[:, :1] + 1.0
        dv = jnp.broadcast_to(lax.rsqrt(d), (_BLK, 128))
        dv_ref[...] = dv
        z = z_ref[...]
        za_ref[...] = z[:, :128] * dv
        zb_ref[...] = z[:, 128:] * dv

    return pl.pallas_call(
        body,
        grid=(N // _BLK,),
        in_specs=[pl.BlockSpec((_BLK, 128), lambda i: (i, 0)),
                  pl.BlockSpec((_BLK, 128), lambda i: (i, 0)),
                  pl.BlockSpec((_BLK, H), lambda i: (i, 0))],
        out_specs=[pl.BlockSpec((_BLK, 128), lambda i: (i, 0))] * 3,
        out_shape=[jax.ShapeDtypeStruct((N, 128), jnp.float32)] * 3,
    )(degA, degB, z0)


def _fused_call(accA, accB, ztA, ztB, dinv, b2d, g2d, beta2d, Wn):
    def body(aa, ab, za, zb, dv_ref, b_ref, g_ref, be_ref, w_ref,
             oa_ref, ob_ref):
        dv = dv_ref[...]
        sc = g_ref[...] * (1.0 / jnp.sqrt(1.0 + 1e-5))
        oA = jnp.maximum(dv * (aa[...] + za[...]) + b_ref[:, :128], 0.0)
        oB = jnp.maximum(dv * (ab[...] + zb[...]) + b_ref[:, 128:], 0.0)
        h = jnp.concatenate([oA, oB], axis=1) * sc + be_ref[...]
        z = jnp.dot(h.astype(jnp.bfloat16), w_ref[...],
                    preferred_element_type=jnp.float32)
        oa_ref[...] = z[:, :128] * dv
        ob_ref[...] = z[:, 128:] * dv

    return pl.pallas_call(
        body,
        grid=(N // _BLK,),
        in_specs=[pl.BlockSpec((_BLK, 128), lambda i: (i, 0))] * 5
        + [pl.BlockSpec((1, H), lambda i: (0, 0))] * 3
        + [pl.BlockSpec((H, H), lambda i: (0, 0))],
        out_specs=[pl.BlockSpec((_BLK, 128), lambda i: (i, 0))] * 2,
        out_shape=[jax.ShapeDtypeStruct((N, 128), jnp.float32)] * 2,
    )(accA, accB, ztA, ztB, dinv, b2d, g2d, beta2d, Wn)


def _tail_call(accA, accB, ztA, ztB, dinv, b2d, Wp1, bp1_2d, Wp2, bp2_2d):
    grid_n = N // _BLK

    def body(aa, ab, za, zb, dv_ref, b_ref, w1_ref, b1_ref, w2_ref, b2_ref,
             o_ref, ssum, smax):
        i = pl.program_id(0)
        dv = dv_ref[...]
        oA = jnp.maximum(dv * (aa[...] + za[...]) + b_ref[:, :128], 0.0)
        oB = jnp.maximum(dv * (ab[...] + zb[...]) + b_ref[:, 128:], 0.0)
        x = jnp.concatenate([oA, oB], axis=1)
        xr = x.reshape(_BLK // 8, 8, H)
        ps = jnp.sum(xr, axis=0)
        pm = jnp.max(xr, axis=0)

        @pl.when(i == 0)
        def _():
            ssum[...] = ps
            smax[...] = pm

        @pl.when(i > 0)
        def _():
            ssum[...] = ssum[...] + ps
            smax[...] = jnp.maximum(smax[...], pm)

        @pl.when(i == grid_n - 1)
        def _():
            tot = jnp.sum(ssum[...], axis=0, keepdims=True) * (1.0 / N)
            mx = jnp.max(smax[...], axis=0, keepdims=True)
            graph = jnp.concatenate([tot, mx], axis=1)
            z = jnp.maximum(
                jnp.dot(graph, w1_ref[...], preferred_element_type=jnp.float32)
                + b1_ref[...], 0.0)
            o_ref[...] = (
                jnp.dot(z, w2_ref[...], preferred_element_type=jnp.float32)
                + b2_ref[...])

    return pl.pallas_call(
        body,
        grid=(grid_n,),
        in_specs=[pl.BlockSpec((_BLK, 128), lambda i: (i, 0))] * 5
        + [pl.BlockSpec((1, H), lambda i: (0, 0)),
           pl.BlockSpec((2 * H, H), lambda i: (0, 0)),
           pl.BlockSpec((1, H), lambda i: (0, 0)),
           pl.BlockSpec((H, OUT), lambda i: (0, 0)),
           pl.BlockSpec((1, OUT), lambda i: (0, 0))],
        out_specs=pl.BlockSpec((1, OUT), lambda i: (0, 0)),
        out_shape=jax.ShapeDtypeStruct((1, OUT), jnp.float32),
        scratch_shapes=[pltpu.VMEM((8, H), jnp.float32),
                        pltpu.VMEM((8, H), jnp.float32)],
    )(accA, accB, ztA, ztB, dinv, b2d, Wp1, bp1_2d, Wp2, bp2_2d)


def kernel(x, edge_index, W0, b0, W1, b1, W2, b2, g0, beta0, g1, beta1,
           Wp1, bp1, Wp2, bp2):
    src = edge_index[0]
    dst = edge_index[1]
    srcr = src.reshape(NS, NB, B)
    dstr = dst.reshape(NS, NB, B)
    dstd = dst.reshape(NC * NS, NBD, BD)
    zslab = jnp.zeros((SLAB, 128), jnp.float32)
    ones_rows = jnp.ones((BD, 128), jnp.float32)

    degA, degB = _deg_call(dstd, ones_rows, zslab)
    z0 = _mm0_call(x, W0.astype(jnp.bfloat16))
    dinv, ztA, ztB = _scale0_call(degA, degB, z0)

    accA, accB = _msgpass_call(ztA, ztB, srcr, dstr, zslab)
    ztA, ztB = _fused_call(accA, accB, ztA, ztB, dinv,
                           b0.reshape(1, H), g0.reshape(1, H),
                           beta0.reshape(1, H), W1.astype(jnp.bfloat16))
    accA, accB = _msgpass_call(ztA, ztB, srcr, dstr, zslab)
    ztA, ztB = _fused_call(accA, accB, ztA, ztB, dinv,
                           b1.reshape(1, H), g1.reshape(1, H),
                           beta1.reshape(1, H), W2.astype(jnp.bfloat16))
    accA, accB = _msgpass_call(ztA, ztB, srcr, dstr, zslab)

    return _tail_call(accA, accB, ztA, ztB, dinv, b2.reshape(1, H),
                      Wp1, bp1.reshape(1, H), Wp2, bp2.reshape(1, OUT))


# speedup vs baseline: 1.1507x; 1.0640x over previous
import dataclasses
import functools

import jax
import jax.numpy as jnp
from jax import lax
from jax.experimental import pallas as pl
from jax.experimental.pallas import tpu as pltpu
from jax.experimental.pallas import tpu_sc as plsc

N = 10000
E = 160000
D = 256
H = 256
OUT = 128

NC = 2
NS = 16
NP = 10240
SLAB = NP // NS
EPT = E // NS
B = 125
NB = EPT // B
KB = 16
NCH = NB // KB
EPT_DEG = E // (NC * NS)
BD = 125
NBD = EPT_DEG // BD


def _vector_mesh():
    return plsc.VectorSubcoreMesh(core_axis_name="c", subcore_axis_name="s")


EPD = 5120
NQ = EPD // 16


def _deg_call(dstp):
    cp = pltpu.CompilerParams()
    if "needs_layout_passes" in pltpu.CompilerParams.__dataclass_fields__:
        cp = dataclasses.replace(cp, needs_layout_passes=False)

    @functools.partial(
        pl.kernel,
        out_type=[jax.ShapeDtypeStruct((NP, 16), jnp.float32)] * 2,
        mesh=_vector_mesh(),
        compiler_params=cp,
        scratch_types=[
            pltpu.VMEM((EPD,), jnp.int32),
            pltpu.VMEM((NP,), jnp.float32),
            pltpu.VMEM((NS, SLAB), jnp.float32),
            pltpu.VMEM((SLAB, 16), jnp.float32),
            pltpu.VMEM_SHARED((NS, NP), jnp.float32),
        ],
    )
    def k(dstp_hbm, outA_hbm, outB_hbm, idx_v, degloc, redbuf, outbuf,
          stage_sh):
        c = lax.axis_index("c")
        s = lax.axis_index("s")
        ones16 = jnp.full((16,), 1.0, jnp.float32)
        zeros16f = jnp.zeros((16,), jnp.float32)
        zeros16i = jnp.zeros((16,), jnp.int32)
        iota16 = lax.iota(jnp.int32, 16)

        def run(chunk, out_hbm):
            pltpu.sync_copy(dstp_hbm.at[chunk], idx_v)

            @pl.loop(0, NP // 16, step=8)
            def _(i):
                for u in range(8):
                    degloc[pl.ds((i + u) * 16, 16)] = zeros16f

            @pl.loop(0, NQ, step=8)
            def _(i):
                for u in range(8):
                    iv = idx_v[pl.ds((i + u) * 16, 16)]
                    plsc.addupdate_scatter(degloc, [iv], ones16)

            pltpu.sync_copy(degloc, stage_sh.at[s])
            plsc.subcore_barrier()
            for t in range(NS):
                pltpu.sync_copy(stage_sh.at[t].at[pl.ds(s * SLAB, SLAB)],
                                redbuf.at[t])

            @pl.loop(0, SLAB // 16)
            def _(kq):
                acc = redbuf[0, pl.ds(kq * 16, 16)]
                for t in range(1, NS):
                    acc = acc + redbuf[t, pl.ds(kq * 16, 16)]
                plsc.store_scatter(outbuf, [kq * 16 + iota16, zeros16i], acc)

            pltpu.sync_copy(outbuf, out_hbm.at[pl.ds(s * SLAB, SLAB)])

        @pl.when(c == 0)
        def _():
            run(s, outA_hbm)

        @pl.when(c == 1)
        def _():
            run(s + NS, outB_hbm)

    return k(dstp)


def _msgpass_call(ztA, ztB, srcr, dstr, zslab):
    @functools.partial(
        pl.kernel,
        out_type=[jax.ShapeDtypeStruct((NP, 128), jnp.float32)] * 2,
        mesh=_vector_mesh(),
        scratch_types=[
            pltpu.VMEM((KB, B), jnp.int32),
            pltpu.VMEM((KB, B), jnp.int32),
            pltpu.VMEM((B, 128), jnp.float32),
            pltpu.VMEM((B, 128), jnp.float32),
            pltpu.VMEM_SHARED((NP, 128), jnp.float32),
            pltpu.SemaphoreType.DMA,
            pltpu.SemaphoreType.DMA,
            pltpu.SemaphoreType.DMA,
        ],
    )
    def k(ztA_hbm, ztB_hbm, src_hbm, dst_hbm, z_hbm, outA_hbm, outB_hbm,
          idx_s, idx_d, rows0, rows1, acc_sh, sem0, sem1, semz):
        c = lax.axis_index("c")
        s = lax.axis_index("s")

        def run(zt_hbm, out_hbm):
            zdesc = pltpu.async_copy(z_hbm, acc_sh.at[pl.ds(s * SLAB, SLAB)],
                                     semz)

            def start_g(j, buf, sem):
                pltpu.make_async_copy(zt_hbm.at[idx_s.at[j]], buf, sem).start()

            def wait_g(buf, sem):
                pltpu.make_async_copy(zt_hbm.at[idx_s.at[0]], buf, sem).wait()

            @pl.loop(0, NCH)
            def _(ch):
                pltpu.sync_copy(src_hbm.at[s].at[pl.ds(ch * KB, KB)], idx_s)
                pltpu.sync_copy(dst_hbm.at[s].at[pl.ds(ch * KB, KB)], idx_d)
                start_g(0, rows0, sem0)

                @pl.when(ch == 0)
                def _():
                    zdesc.wait()
                    plsc.subcore_barrier()

                @pl.loop(0, KB // 2)
                def _(j2):
                    j = 2 * j2
                    wait_g(rows0, sem0)
                    start_g(j + 1, rows1, sem1)
                    pltpu.sync_copy(rows0, acc_sh.at[idx_d.at[j]], add=True)
                    wait_g(rows1, sem1)

                    @pl.when(j + 2 < KB)
                    def _():
                        start_g(j + 2, rows0, sem0)

                    pltpu.sync_copy(rows1, acc_sh.at[idx_d.at[j + 1]], add=True)

            plsc.subcore_barrier()
            pltpu.sync_copy(acc_sh.at[pl.ds(s * SLAB, SLAB)],
                            out_hbm.at[pl.ds(s * SLAB, SLAB)])

        @pl.when(c == 0)
        def _():
            run(ztA_hbm, outA_hbm)

        @pl.when(c == 1)
        def _():
            run(ztB_hbm, outB_hbm)

    return k(ztA, ztB, srcr, dstr, zslab)


_BLK = 1000


def _mm0_call(x, W0):
    def body(h_ref, w_ref, z_ref):
        z_ref[...] = jnp.dot(h_ref[...].astype(jnp.bfloat16), w_ref[...],
                             preferred_element_type=jnp.float32)

    return pl.pallas_call(
        body,
        grid=(N // _BLK,),
        in_specs=[pl.BlockSpec((_BLK, D), lambda i: (i, 0)),
                  pl.BlockSpec((D, H), lambda i: (0, 0))],
        out_specs=pl.BlockSpec((_BLK, H), lambda i: (i, 0)),
        out_shape=jax.ShapeDtypeStruct((N, H), jnp.float32),
    )(x, W0)


def _scale0_call(degA, degB, z0):
    def body(da_ref, db_ref, z_ref, dv_ref, za_ref, zb_ref):
        d = da_ref[:, :1] + db_ref[:, :1] + 1.0
        dv = jnp.broadcast_to(lax.rsqrt(d), (_BLK, 128))
        dv_ref[...] = dv
        z = z_ref[...]
        za_ref[...] = z[:, :128] * dv
        zb_ref[...] = z[:, 128:] * dv

    return pl.pallas_call(
        body,
        grid=(N // _BLK,),
        in_specs=[pl.BlockSpec((_BLK, 16), lambda i: (i, 0)),
                  pl.BlockSpec((_BLK, 16), lambda i: (i, 0)),
                  pl.BlockSpec((_BLK, H), lambda i: (i, 0))],
        out_specs=[pl.BlockSpec((_BLK, 128), lambda i: (i, 0))] * 3,
        out_shape=[jax.ShapeDtypeStruct((N, 128), jnp.float32)] * 3,
    )(degA, degB, z0)


def _fused_call(accA, accB, ztA, ztB, dinv, b2d, g2d, beta2d, Wn):
    def body(aa, ab, za, zb, dv_ref, b_ref, g_ref, be_ref, w_ref,
             oa_ref, ob_ref):
        dv = dv_ref[...]
        sc = g_ref[...] * (1.0 / jnp.sqrt(1.0 + 1e-5))
        oA = jnp.maximum(dv * (aa[...] + za[...]) + b_ref[:, :128], 0.0)
        oB = jnp.maximum(dv * (ab[...] + zb[...]) + b_ref[:, 128:], 0.0)
        h = jnp.concatenate([oA, oB], axis=1) * sc + be_ref[...]
        z = jnp.dot(h.astype(jnp.bfloat16), w_ref[...],
                    preferred_element_type=jnp.float32)
        oa_ref[...] = z[:, :128] * dv
        ob_ref[...] = z[:, 128:] * dv

    return pl.pallas_call(
        body,
        grid=(N // _BLK,),
        in_specs=[pl.BlockSpec((_BLK, 128), lambda i: (i, 0))] * 5
        + [pl.BlockSpec((1, H), lambda i: (0, 0))] * 3
        + [pl.BlockSpec((H, H), lambda i: (0, 0))],
        out_specs=[pl.BlockSpec((_BLK, 128), lambda i: (i, 0))] * 2,
        out_shape=[jax.ShapeDtypeStruct((N, 128), jnp.float32)] * 2,
    )(accA, accB, ztA, ztB, dinv, b2d, g2d, beta2d, Wn)


def _tail_call(accA, accB, ztA, ztB, dinv, b2d, Wp1, bp1_2d, Wp2, bp2_2d):
    grid_n = N // _BLK

    def body(aa, ab, za, zb, dv_ref, b_ref, w1_ref, b1_ref, w2_ref, b2_ref,
             o_ref, ssum, smax):
        i = pl.program_id(0)
        dv = dv_ref[...]
        oA = jnp.maximum(dv * (aa[...] + za[...]) + b_ref[:, :128], 0.0)
        oB = jnp.maximum(dv * (ab[...] + zb[...]) + b_ref[:, 128:], 0.0)
        x = jnp.concatenate([oA, oB], axis=1)
        xr = x.reshape(_BLK // 8, 8, H)
        ps = jnp.sum(xr, axis=0)
        pm = jnp.max(xr, axis=0)

        @pl.when(i == 0)
        def _():
            ssum[...] = ps
            smax[...] = pm

        @pl.when(i > 0)
        def _():
            ssum[...] = ssum[...] + ps
            smax[...] = jnp.maximum(smax[...], pm)

        @pl.when(i == grid_n - 1)
        def _():
            tot = jnp.sum(ssum[...], axis=0, keepdims=True) * (1.0 / N)
            mx = jnp.max(smax[...], axis=0, keepdims=True)
            graph = jnp.concatenate([tot, mx], axis=1)
            z = jnp.maximum(
                jnp.dot(graph, w1_ref[...], preferred_element_type=jnp.float32)
                + b1_ref[...], 0.0)
            o_ref[...] = (
                jnp.dot(z, w2_ref[...], preferred_element_type=jnp.float32)
                + b2_ref[...])

    return pl.pallas_call(
        body,
        grid=(grid_n,),
        in_specs=[pl.BlockSpec((_BLK, 128), lambda i: (i, 0))] * 5
        + [pl.BlockSpec((1, H), lambda i: (0, 0)),
           pl.BlockSpec((2 * H, H), lambda i: (0, 0)),
           pl.BlockSpec((1, H), lambda i: (0, 0)),
           pl.BlockSpec((H, OUT), lambda i: (0, 0)),
           pl.BlockSpec((1, OUT), lambda i: (0, 0))],
        out_specs=pl.BlockSpec((1, OUT), lambda i: (0, 0)),
        out_shape=jax.ShapeDtypeStruct((1, OUT), jnp.float32),
        scratch_shapes=[pltpu.VMEM((8, H), jnp.float32),
                        pltpu.VMEM((8, H), jnp.float32)],
    )(accA, accB, ztA, ztB, dinv, b2d, Wp1, bp1_2d, Wp2, bp2_2d)


def kernel(x, edge_index, W0, b0, W1, b1, W2, b2, g0, beta0, g1, beta1,
           Wp1, bp1, Wp2, bp2):
    src = edge_index[0]
    dst = edge_index[1]
    srcr = src.reshape(NS, NB, B)
    dstr = dst.reshape(NS, NB, B)
    dstp = jnp.concatenate(
        [dst, jnp.full((NC * NS * EPD - E,), NP - 8, jnp.int32)]
    ).reshape(NC * NS, EPD)
    zslab = jnp.zeros((SLAB, 128), jnp.float32)

    degA, degB = _deg_call(dstp)
    z0 = _mm0_call(x, W0.astype(jnp.bfloat16))
    dinv, ztA, ztB = _scale0_call(degA, degB, z0)

    accA, accB = _msgpass_call(ztA, ztB, srcr, dstr, zslab)
    ztA, ztB = _fused_call(accA, accB, ztA, ztB, dinv,
                           b0.reshape(1, H), g0.reshape(1, H),
                           beta0.reshape(1, H), W1.astype(jnp.bfloat16))
    accA, accB = _msgpass_call(ztA, ztB, srcr, dstr, zslab)
    ztA, ztB = _fused_call(accA, accB, ztA, ztB, dinv,
                           b1.reshape(1, H), g1.reshape(1, H),
                           beta1.reshape(1, H), W2.astype(jnp.bfloat16))
    accA, accB = _msgpass_call(ztA, ztB, srcr, dstr, zslab)

    return _tail_call(accA, accB, ztA, ztB, dinv, b2.reshape(1, H),
                      Wp1, bp1.reshape(1, H), Wp2, bp2.reshape(1, OUT))
